# baseline (device time: 89375 ns/iter reference)
import jax
import jax.numpy as jnp
from jax import lax
from jax.experimental import pallas as pl
from jax.experimental.pallas import tpu as pltpu

T = 2048
D = 1024


def kernel(x, dest):
    order = jnp.argsort(dest, stable=True)
    s = x.astype(jnp.bfloat16)[order]

    def body(s_ref, out_ref, send_sem, recv_sem):
        mx = lax.axis_index("x")
        my = lax.axis_index("y")
        mz = lax.axis_index("z")
        rdma = pltpu.make_async_remote_copy(
            src_ref=s_ref,
            dst_ref=out_ref,
            send_sem=send_sem,
            recv_sem=recv_sem,
            device_id=(mx, 1 - my, mz),
            device_id_type=pl.DeviceIdType.MESH,
        )
        rdma.start()
        rdma.wait()

    theirs = pl.pallas_call(
        body,
        out_shape=jax.ShapeDtypeStruct((T, D), jnp.bfloat16),
        in_specs=[pl.BlockSpec(memory_space=pltpu.VMEM)],
        out_specs=pl.BlockSpec(memory_space=pltpu.VMEM),
        scratch_shapes=[
            pltpu.SemaphoreType.DMA,
            pltpu.SemaphoreType.DMA,
        ],
    )(s)

    my_y = lax.axis_index("y")
    c0 = jnp.sum((dest == 0).astype(jnp.int32))
    i = jnp.arange(T)
    from_mine = jnp.where(my_y == 0, i < c0, i >= c0)
    theirs_idx = jnp.where(my_y == 0, i - c0, (T - c0) + i)
    theirs_idx = jnp.clip(theirs_idx, 0, T - 1)
    return jnp.where(from_mine[:, None], s, theirs[theirs_idx])


# device time: 75712 ns/iter; 1.1805x vs baseline; 1.1805x over previous
import jax
import jax.numpy as jnp
from jax import lax
from jax.experimental import pallas as pl
from jax.experimental.pallas import tpu as pltpu

T = 2048
D = 1024
R = 64
MAX_CHUNKS = T // R


def kernel(x, dest):
    my_y = lax.axis_index("y")
    order = jnp.argsort(dest, stable=True)
    s = x.astype(jnp.bfloat16)[order]

    c0 = jnp.sum((dest == 0).astype(jnp.int32))
    K = jnp.where(my_y == 0, T - c0, c0).astype(jnp.int32)
    base = jnp.where(my_y == 0, c0, 0).astype(jnp.int32)
    send_buf = jnp.roll(s, -base, axis=0)
    n_chunks = (K + R - 1) // R
    scal = n_chunks.reshape(1).astype(jnp.int32)

    def body(scal_ref, send_ref, stage_ref, send_sems, recv_sems):
        mx = lax.axis_index("x")
        my = lax.axis_index("y")
        mz = lax.axis_index("z")
        nbr = (mx, 1 - my, mz)
        nc = scal_ref[0]

        for j in range(MAX_CHUNKS):
            @pl.when(j < nc)
            def _():
                rdma = pltpu.make_async_remote_copy(
                    src_ref=send_ref.at[pl.ds(j * R, R), :],
                    dst_ref=stage_ref.at[pl.ds(j * R, R), :],
                    send_sem=send_sems.at[j],
                    recv_sem=recv_sems.at[j],
                    device_id=nbr,
                    device_id_type=pl.DeviceIdType.MESH,
                )
                rdma.start()

        for j in range(MAX_CHUNKS):
            @pl.when(j < nc)
            def _():
                rdma = pltpu.make_async_remote_copy(
                    src_ref=send_ref.at[pl.ds(j * R, R), :],
                    dst_ref=stage_ref.at[pl.ds(j * R, R), :],
                    send_sem=send_sems.at[j],
                    recv_sem=recv_sems.at[j],
                    device_id=nbr,
                    device_id_type=pl.DeviceIdType.MESH,
                )
                rdma.wait()

    staging = pl.pallas_call(
        body,
        out_shape=jax.ShapeDtypeStruct((T, D), jnp.bfloat16),
        in_specs=[
            pl.BlockSpec(memory_space=pltpu.SMEM),
            pl.BlockSpec(memory_space=pltpu.VMEM),
        ],
        out_specs=pl.BlockSpec(memory_space=pltpu.VMEM),
        scratch_shapes=[
            pltpu.SemaphoreType.DMA((MAX_CHUNKS,)),
            pltpu.SemaphoreType.DMA((MAX_CHUNKS,)),
        ],
    )(scal, send_buf)

    i = jnp.arange(T)
    from_mine = jnp.where(my_y == 0, i < c0, i >= c0)
    stage_idx = jnp.where(my_y == 0, i - c0, i)
    stage_idx = jnp.clip(stage_idx, 0, T - 1)
    return jnp.where(from_mine[:, None], s, staging[stage_idx])


# device time: 65345 ns/iter; 1.3677x vs baseline; 1.1587x over previous
import jax
import jax.numpy as jnp
from jax import lax
from jax.experimental import pallas as pl
from jax.experimental.pallas import tpu as pltpu

T = 2048
D = 1024
R = 64
MAX_CHUNKS = T // R


def kernel(x, dest):
    my_y = lax.axis_index("y")
    order = jnp.argsort(dest, stable=True)
    s = x.astype(jnp.bfloat16)[order]

    c0 = jnp.sum((dest == 0).astype(jnp.int32))
    K = jnp.where(my_y == 0, T - c0, c0)
    base = jnp.where(my_y == 0, c0, 0)
    roff = jnp.where(my_y == 0, c0, 0)
    shift_send = (T - base) % T
    n_chunks = (K + R - 1) // R
    scal = jnp.stack([n_chunks, shift_send, roff, c0]).astype(jnp.int32)

    def body(scal_ref, s_ref, out_ref, send_buf, stage, send_sems, recv_sems):
        mx = lax.axis_index("x")
        my = lax.axis_index("y")
        mz = lax.axis_index("z")
        nbr = (mx, 1 - my, mz)
        nc = scal_ref[0]
        shift = scal_ref[1]
        ro = scal_ref[2]
        c0s = scal_ref[3]

        send_buf[:, :] = pltpu.roll(s_ref[:, :], shift, 0)

        for j in range(MAX_CHUNKS):
            @pl.when(j < nc)
            def _():
                rdma = pltpu.make_async_remote_copy(
                    src_ref=send_buf.at[pl.ds(j * R, R), :],
                    dst_ref=stage.at[pl.ds(j * R, R), :],
                    send_sem=send_sems.at[j],
                    recv_sem=recv_sems.at[j],
                    device_id=nbr,
                    device_id_type=pl.DeviceIdType.MESH,
                )
                rdma.start()

        for j in range(MAX_CHUNKS):
            @pl.when(j < nc)
            def _():
                rdma = pltpu.make_async_remote_copy(
                    src_ref=send_buf.at[pl.ds(j * R, R), :],
                    dst_ref=stage.at[pl.ds(j * R, R), :],
                    send_sem=send_sems.at[j],
                    recv_sem=recv_sems.at[j],
                    device_id=nbr,
                    device_id_type=pl.DeviceIdType.MESH,
                )
                rdma.wait()

        rolled = pltpu.roll(stage[:, :], ro, 0)
        rows = lax.broadcasted_iota(jnp.int32, (T, D), 0)
        lo = my * c0s
        hi = my * T + (1 - my) * c0s
        from_mine = (rows >= lo) & (rows < hi)
        out_ref[:, :] = jnp.where(from_mine, s_ref[:, :], rolled)

    return pl.pallas_call(
        body,
        out_shape=jax.ShapeDtypeStruct((T, D), jnp.bfloat16),
        in_specs=[
            pl.BlockSpec(memory_space=pltpu.SMEM),
            pl.BlockSpec(memory_space=pltpu.VMEM),
        ],
        out_specs=pl.BlockSpec(memory_space=pltpu.VMEM),
        scratch_shapes=[
            pltpu.VMEM((T, D), jnp.bfloat16),
            pltpu.VMEM((T, D), jnp.bfloat16),
            pltpu.SemaphoreType.DMA((MAX_CHUNKS,)),
            pltpu.SemaphoreType.DMA((MAX_CHUNKS,)),
        ],
    )(scal, s)


# device time: 60003 ns/iter; 1.4895x vs baseline; 1.0890x over previous
import jax
import jax.numpy as jnp
from jax import lax
from jax.experimental import pallas as pl
from jax.experimental.pallas import tpu as pltpu

T = 2048
D = 1024
R = 64
MAX_CHUNKS = T // R


def kernel(x, dest):
    my_y = lax.axis_index("y")
    order = jnp.argsort(dest, stable=True)
    s = x.astype(jnp.bfloat16)[order].reshape(T, 8, 128)

    c0 = jnp.sum((dest == 0).astype(jnp.int32))
    K = jnp.where(my_y == 0, T - c0, c0)
    base = jnp.where(my_y == 0, c0, 0)
    doff = jnp.where(my_y == 0, 0, T - K)
    klo = jnp.where(my_y == 0, 0, c0)
    kn = T - K
    nc = (K + R - 1) // R
    scal = jnp.stack([nc, base, doff, K, klo, kn]).astype(jnp.int32)

    def body(scal_ref, s_ref, out_ref, send_sems, recv_sems):
        mx = lax.axis_index("x")
        my = lax.axis_index("y")
        mz = lax.axis_index("z")
        nbr = (mx, 1 - my, mz)
        nc_ = scal_ref[0]
        base_ = scal_ref[1]
        doff_ = scal_ref[2]
        k_ = scal_ref[3]
        klo_ = scal_ref[4]
        kn_ = scal_ref[5]

        for j in range(MAX_CHUNKS):
            @pl.when(j < nc_)
            def _():
                o = jnp.maximum(jnp.minimum(j * R, k_ - R), 0)
                rdma = pltpu.make_async_remote_copy(
                    src_ref=s_ref.at[pl.ds(base_ + o, R), :, :],
                    dst_ref=out_ref.at[pl.ds(doff_ + o, R), :, :],
                    send_sem=send_sems.at[j],
                    recv_sem=recv_sems.at[j],
                    device_id=nbr,
                    device_id_type=pl.DeviceIdType.MESH,
                )
                rdma.start()

        for j in range(MAX_CHUNKS):
            @pl.when(j < nc_)
            def _():
                o = jnp.maximum(jnp.minimum(j * R, k_ - R), 0)
                rdma = pltpu.make_async_remote_copy(
                    src_ref=s_ref.at[pl.ds(base_ + o, R), :, :],
                    dst_ref=out_ref.at[pl.ds(doff_ + o, R), :, :],
                    send_sem=send_sems.at[j],
                    recv_sem=recv_sems.at[j],
                    device_id=nbr,
                    device_id_type=pl.DeviceIdType.MESH,
                )
                rdma.wait()

        rows = lax.broadcasted_iota(jnp.int32, (T, 8, 128), 0)
        kept = (rows >= klo_) & (rows < klo_ + kn_)
        out_ref[:, :, :] = jnp.where(kept, s_ref[:, :, :], out_ref[:, :, :])

    return pl.pallas_call(
        body,
        out_shape=jax.ShapeDtypeStruct((T, 8, 128), jnp.bfloat16),
        in_specs=[
            pl.BlockSpec(memory_space=pltpu.SMEM),
            pl.BlockSpec(memory_space=pltpu.VMEM),
        ],
        out_specs=pl.BlockSpec(memory_space=pltpu.VMEM),
        scratch_shapes=[
            pltpu.SemaphoreType.DMA((MAX_CHUNKS,)),
            pltpu.SemaphoreType.DMA((MAX_CHUNKS,)),
        ],
    )(scal, s).reshape(T, D)
